# Initial kernel scaffold; baseline (speedup 1.0000x reference)
import jax
import jax.numpy as jnp
from jax.experimental import pallas as pl


def kernel(x, edge_index, W, b):
    raise NotImplementedError("write your pallas kernel here")


# speedup vs baseline: 21.1010x; 21.1010x over previous
import functools

import jax
import jax.numpy as jnp
from jax import lax
from jax.experimental import pallas as pl
from jax.experimental.pallas import tpu as pltpu
from jax.experimental.pallas import tpu_sc as plsc


def _dg(x, idx):
    return lax.gather(
        x, idx[:, None],
        dimension_numbers=lax.GatherDimensionNumbers(
            offset_dims=(), collapsed_slice_dims=(0,), start_index_map=(0,)),
        slice_sizes=(1,), mode=lax.GatherScatterMode.PROMISE_IN_BOUNDS)


def _zero_ref(ref, n, dtype=jnp.float32):
    zero = jnp.zeros((16,), dtype)
    def body(i, _):
        ref[pl.ds(i * 16, 16)] = zero
        return 0
    lax.fori_loop(0, n // 16, body, 0)


def _fill_ref(ref, n, value, dtype=jnp.float32):
    v = jnp.full((16,), value, dtype)
    def body(i, _):
        ref[pl.ds(i * 16, 16)] = v
        return 0
    lax.fori_loop(0, n // 16, body, 0)


def _make_sc_kernels(N, E, D, K):
    NPAD = 10240
    TOT = E + N
    mesh = plsc.VectorSubcoreMesh(core_axis_name="c", subcore_axis_name="s")
    EW16 = E // 16
    EW32 = E // 32

    @functools.partial(
        pl.kernel, mesh=mesh,
        compiler_params=pltpu.CompilerParams(use_tc_tiling_on_sc=False, needs_layout_passes=False),
        out_type=jax.ShapeDtypeStruct((N,), jnp.float32),
        scratch_types=[
            pltpu.VMEM((EW16,), jnp.int32),
            pltpu.VMEM((EW16,), jnp.float32),
            pltpu.VMEM((NPAD,), jnp.float32),
            pltpu.VMEM_SHARED((NPAD,), jnp.float32),
        ],
    )
    def deg_kernel(col_hbm, out_hbm, idx_v, ones_v, zbuf_v, hist_sp):
        cid = lax.axis_index("c")
        sid = lax.axis_index("s")

        @pl.when(cid == 0)
        def _():
            @pl.when(sid == 0)
            def _():
                _zero_ref(zbuf_v, NPAD)
                pltpu.sync_copy(zbuf_v, hist_sp)
            plsc.subcore_barrier()
            pltpu.sync_copy(col_hbm.at[pl.ds(sid * EW16, EW16)], idx_v)
            _fill_ref(ones_v, EW16, 1.0)
            pltpu.sync_copy(ones_v, hist_sp.at[idx_v], add=True)
            plsc.subcore_barrier()

            @pl.when(sid == 0)
            def _():
                pltpu.sync_copy(hist_sp.at[pl.ds(0, N)], zbuf_v.at[pl.ds(0, N)])
                pltpu.sync_copy(zbuf_v.at[pl.ds(0, N)], out_hbm)

    @functools.partial(
        pl.kernel, mesh=mesh,
        compiler_params=pltpu.CompilerParams(use_tc_tiling_on_sc=False, needs_layout_passes=False),
        out_type=jax.ShapeDtypeStruct((E,), jnp.float32),
        scratch_types=[
            pltpu.VMEM((EW32,), jnp.int32),
            pltpu.VMEM((EW32,), jnp.int32),
            pltpu.VMEM((N,), jnp.float32),
            pltpu.VMEM((N,), jnp.float32),
            pltpu.VMEM((EW32,), jnp.float32),
        ],
    )
    def msg_kernel(row_hbm, col_hbm, dinv_hbm, h_hbm, out_hbm,
                   row_v, col_v, dinv_v, h_v, out_v):
        cid = lax.axis_index("c")
        sid = lax.axis_index("s")
        wid = sid * 2 + cid
        base = wid * EW32
        pltpu.sync_copy(row_hbm.at[pl.ds(base, EW32)], row_v)
        pltpu.sync_copy(col_hbm.at[pl.ds(base, EW32)], col_v)
        pltpu.sync_copy(dinv_hbm, dinv_v)
        pltpu.sync_copy(h_hbm, h_v)

        def body(i, _):
            r = row_v[pl.ds(i * 16, 16)]
            c = col_v[pl.ds(i * 16, 16)]
            dr = plsc.load_gather(dinv_v, [r])
            dc = plsc.load_gather(dinv_v, [c])
            hr = plsc.load_gather(h_v, [r])
            out_v[pl.ds(i * 16, 16)] = (dr * dc) * hr
            return 0
        lax.fori_loop(0, EW32 // 16, body, 0)
        pltpu.sync_copy(out_v, out_hbm.at[pl.ds(base, EW32)])

    NW = -(-TOT // 6000)
    PERW, REMW = NW // 16, NW % 16
    MAXLEN = 6000 * (PERW + 1)

    @functools.partial(
        pl.kernel, mesh=mesh,
        compiler_params=pltpu.CompilerParams(use_tc_tiling_on_sc=False, needs_layout_passes=False),
        out_type=jax.ShapeDtypeStruct((N,), jnp.float32),
        scratch_types=[
            pltpu.VMEM((MAXLEN,), jnp.int32),
            pltpu.VMEM((MAXLEN,), jnp.float32),
            pltpu.VMEM((NPAD,), jnp.float32),
            pltpu.VMEM((NPAD,), jnp.float32),
            pltpu.VMEM((NPAD,), jnp.int32),
            pltpu.VMEM_SHARED((NPAD,), jnp.float32),
        ],
    )
    def seg_kernel(ids_hbm, vals_hbm, out_hbm, ids_v, vals_v, acc_v,
                   zbuf_v, idxall_v, acc_sp):
        cid = lax.axis_index("c")
        sid = lax.axis_index("s")
        iota = lax.iota(jnp.int32, 16)
        zf = jnp.zeros((16,), jnp.float32)

        @pl.when(cid == 0)
        def _():
            @pl.when(sid == 0)
            def _():
                _zero_ref(zbuf_v, NPAD)
                pltpu.sync_copy(zbuf_v, acc_sp)
            plsc.subcore_barrier()

            w0 = sid * PERW + jnp.minimum(sid, REMW)
            nwin = PERW + jnp.where(sid < REMW, 1, 0)
            start = w0 * 6000
            length = jnp.minimum(start + nwin * 6000, TOT) - start
            pltpu.sync_copy(
                ids_hbm.at[pl.ds(start, 6000 * PERW)], ids_v.at[pl.ds(0, 6000 * PERW)])
            pltpu.sync_copy(
                vals_hbm.at[pl.ds(start, 6000 * PERW)], vals_v.at[pl.ds(0, 6000 * PERW)])

            @pl.when(sid < REMW)
            def _():
                s2 = jnp.minimum(start + 6000 * PERW, TOT - 6000)
                pltpu.sync_copy(ids_hbm.at[pl.ds(s2, 6000)],
                                ids_v.at[pl.ds(6000 * PERW, 6000)])
                pltpu.sync_copy(vals_hbm.at[pl.ds(s2, 6000)],
                                vals_v.at[pl.ds(6000 * PERW, 6000)])
            _zero_ref(acc_v, NPAD)

            nchunks = length // 16

            def chunk(c, carry_vec):
                cc = c * 16
                ids_c = ids_v[pl.ds(cc, 16)]
                vals_c = vals_v[pl.ds(cc, 16)]
                prev = plsc.load_gather(ids_v, [jnp.maximum(iota + cc - 1, 0)])
                b = (ids_c != prev) | ((iota == 0) & (c == 0))
                bi = b.astype(jnp.int32)
                b0 = _dg(bi, iota * 0)
                emitp = (iota == 0) & (b0 == 1) & (c > 0)
                plsc.addupdate_scatter(acc_v, [prev], carry_vec, mask=emitp)
                cvec = jnp.where((iota == 0) & (b0 == 0), carry_vec, zf)
                x = vals_c + cvec
                bb = bi
                for d in (1, 2, 4, 8):
                    sidx = jnp.maximum(iota - d, 0)
                    xs = _dg(x, sidx)
                    bs = _dg(bb, sidx)
                    can = (iota >= d) & (bb == 0)
                    x = jnp.where(can, x + xs, x)
                    bb = jnp.where(can, bs, bb)
                bnext = _dg(bi, jnp.minimum(iota + 1, 15))
                emask = (iota < 15) & (bnext == 1)
                plsc.addupdate_scatter(acc_v, [ids_c], x, mask=emask)
                x15 = _dg(x, iota * 0 + 15)
                return jnp.where(iota == 0, x15, zf)

            carry = lax.fori_loop(0, nchunks, chunk, zf)
            idl = plsc.load_gather(ids_v, [iota * 0 + (length - 1)])
            plsc.addupdate_scatter(acc_v, [idl], carry, mask=(iota == 0))

            def ibody(i, _):
                idxall_v[pl.ds(i * 16, 16)] = iota + i * 16
                return 0
            lax.fori_loop(0, NPAD // 16, ibody, 0)
            pltpu.sync_copy(acc_v, acc_sp.at[idxall_v], add=True)
            plsc.subcore_barrier()

            @pl.when(sid == 0)
            def _():
                pltpu.sync_copy(acc_sp.at[pl.ds(0, N)], zbuf_v.at[pl.ds(0, N)])
                pltpu.sync_copy(zbuf_v.at[pl.ds(0, N)], out_hbm)

    PW = NPAD // 16

    @functools.partial(
        pl.kernel, mesh=mesh,
        compiler_params=pltpu.CompilerParams(use_tc_tiling_on_sc=False, needs_layout_passes=False),
        out_type=[jax.ShapeDtypeStruct((NPAD,), jnp.int32),
                  jax.ShapeDtypeStruct((E,), jnp.float32)],
        scratch_types=[
            pltpu.VMEM((PW,), jnp.int32),
            pltpu.VMEM((PW,), jnp.int32),
            pltpu.VMEM((EW32,), jnp.int32),
            pltpu.VMEM((EW32,), jnp.int32),
            pltpu.VMEM((N,), jnp.float32),
            pltpu.VMEM((EW32,), jnp.float32),
            pltpu.VMEM((NPAD,), jnp.int32),
            pltpu.VMEM_SHARED((NPAD,), jnp.int32),
        ],
    )
    def final_kernel(rank_hbm, row_hbm, col_hbm, mask_hbm,
                     perm_hbm, ew_hbm,
                     rank_v, val_v, row_v, col_v, mask_v, ew_v, pbuf_v,
                     perm_sp):
        cid = lax.axis_index("c")
        sid = lax.axis_index("s")
        wid = sid * 2 + cid
        iota = lax.iota(jnp.int32, 16)

        base = wid * EW32
        pltpu.sync_copy(row_hbm.at[pl.ds(base, EW32)], row_v)
        pltpu.sync_copy(col_hbm.at[pl.ds(base, EW32)], col_v)
        pltpu.sync_copy(mask_hbm, mask_v)

        def body(i, _):
            r = row_v[pl.ds(i * 16, 16)]
            c = col_v[pl.ds(i * 16, 16)]
            mr = plsc.load_gather(mask_v, [r])
            mc = plsc.load_gather(mask_v, [c])
            s = mr + mc
            ew_v[pl.ds(i * 16, 16)] = jnp.where(
                s == 2.0, jnp.full((16,), 1.0, jnp.float32),
                jnp.zeros((16,), jnp.float32))
            return 0
        lax.fori_loop(0, EW32 // 16, body, 0)
        pltpu.sync_copy(ew_v, ew_hbm.at[pl.ds(base, EW32)])

        @pl.when(cid == 0)
        def _():
            pbase = sid * PW
            pltpu.sync_copy(rank_hbm.at[pl.ds(pbase, PW)], rank_v)

            def vbody(i, _):
                val_v[pl.ds(i * 16, 16)] = iota + (pbase + i * 16)
                return 0
            lax.fori_loop(0, PW // 16, vbody, 0)
            pltpu.sync_copy(val_v, perm_sp.at[rank_v])
            plsc.subcore_barrier()

            @pl.when(sid == 0)
            def _():
                pltpu.sync_copy(perm_sp, pbuf_v)
                pltpu.sync_copy(pbuf_v, perm_hbm)

    return deg_kernel, msg_kernel, seg_kernel, final_kernel


def _tc_h(x, W, b):
    N, D = x.shape

    def body(x_ref, w_ref, b_ref, o_ref):
        o_ref[...] = jnp.dot(x_ref[...], w_ref[...],
                             preferred_element_type=jnp.float32) + b_ref[0]

    return pl.pallas_call(
        body, out_shape=jax.ShapeDtypeStruct((N, 1), jnp.float32),
    )(x, W, b)


def _tc_prep(deghist, h):
    N = deghist.shape[0]

    def body(d_ref, h_ref, dinv_ref, ms_ref):
        deg = d_ref[...] + 1.0
        dinv = jnp.where(deg > 0, lax.rsqrt(deg), 0.0)
        dinv_ref[...] = dinv
        ms_ref[...] = (dinv * dinv) * h_ref[...]

    return pl.pallas_call(
        body,
        out_shape=[jax.ShapeDtypeStruct((N,), jnp.float32),
                   jax.ShapeDtypeStruct((N,), jnp.float32)],
    )(deghist, h)


def _tc_rank(s, x, K):
    N, D = x.shape
    R = 200
    G = N // R

    def body(s_ref, si_ref, x_ref, x2_ref, rank_ref, mask_ref):
        i = pl.program_id(0)
        score_all = jnp.tanh(s_ref[...])
        bits = lax.bitcast_convert_type(score_all, jnp.int32)
        keys = jnp.where(bits < 0, bits ^ 0x7FFFFFFF, bits)
        kj = keys.reshape(1, N)
        score_i = jnp.tanh(si_ref[...].reshape(R))
        bits_i = lax.bitcast_convert_type(score_i, jnp.int32)
        ki = jnp.where(bits_i < 0, bits_i ^ 0x7FFFFFFF, bits_i).reshape(R, 1)
        jj = lax.broadcasted_iota(jnp.int32, (1, N), 1)
        ii = lax.broadcasted_iota(jnp.int32, (R, 1), 0) + i * R
        ahead = (kj > ki) | ((kj == ki) & (jj < ii))
        rank = jnp.sum(jnp.where(ahead, 1, 0), axis=1)
        rank_ref[...] = rank.reshape(1, 1, R)
        maskf = jnp.where(rank < K, 1.0, 0.0)
        mask_ref[...] = maskf.reshape(1, 1, R)
        x2_ref[...] = (x_ref[...] * score_i[:, None]) * maskf[:, None]

    return pl.pallas_call(
        body,
        grid=(G,),
        in_specs=[
            pl.BlockSpec((N,), lambda i: (0,)),
            pl.BlockSpec((1, 1, R), lambda i: (i, 0, 0)),
            pl.BlockSpec((R, D), lambda i: (i, 0)),
        ],
        out_specs=[
            pl.BlockSpec((R, D), lambda i: (i, 0)),
            pl.BlockSpec((1, 1, R), lambda i: (i, 0, 0)),
            pl.BlockSpec((1, 1, R), lambda i: (i, 0, 0)),
        ],
        out_shape=[jax.ShapeDtypeStruct((N, D), jnp.float32),
                   jax.ShapeDtypeStruct((G, 1, R), jnp.int32),
                   jax.ShapeDtypeStruct((G, 1, R), jnp.float32)],
    )(s, s.reshape(G, 1, R), x)


def kernel(x, edge_index, W, b):
    N, D = x.shape
    E = edge_index.shape[1]
    K = (N + 1) // 2
    NPAD = 10240

    row = edge_index[0]
    col = edge_index[1]
    deg_kernel, msg_kernel, seg_kernel, final_kernel = _make_sc_kernels(
        N, E, D, K)

    deghist = deg_kernel(col)
    h2 = _tc_h(x, W, b)
    h = h2.reshape(-1)
    dinv, msg_self = _tc_prep(deghist, h)
    msg_edges = msg_kernel(row, col, dinv, h)

    loop = jnp.arange(N, dtype=row.dtype)
    col_f = jnp.concatenate([col, loop])
    msg_f = jnp.concatenate([msg_edges, msg_self])
    ids_s, vals_s = lax.sort((col_f, msg_f), dimension=0, num_keys=1,
                             is_stable=False)

    s = seg_kernel(ids_s, vals_s)
    x2, rank3, mask3 = _tc_rank(s, x, K)
    rank = rank3.reshape(N)
    maskf = mask3.reshape(N)

    rank_pad = jnp.concatenate([rank, jnp.arange(N, NPAD, dtype=jnp.int32)])
    perm_pad, ew = final_kernel(rank_pad, row, col, maskf)
    perm = perm_pad[:K]
    batch = jnp.zeros((N,), dtype=jnp.int32)
    return (x2, edge_index, ew, batch, perm)
